# Initial kernel scaffold; baseline (speedup 1.0000x reference)
import jax
import jax.numpy as jnp
from jax.experimental import pallas as pl


def kernel(constraint_features, edge_indices, edge_attrs, variable_features, params):
    raise NotImplementedError("write your pallas kernel here")


# speedup vs baseline: 3.8098x; 3.8098x over previous
import jax
import jax.numpy as jnp
from jax import lax
from jax.experimental import pallas as pl
from jax.experimental.pallas import tpu as pltpu
from jax.experimental.pallas import tpu_sc as plsc

_N = 50000
_E = 800000
_EMB = 64
_H = 32
_NC = 2
_NS = 16
_R0 = 3128
_R15 = _N - 15 * _R0
_ZT = 3072
_EPT = _E // _NS
_CS = 400
_CB = 400
_B = 2000
_NBLK = _N // _B
_EBLK = _E // _B


def _lane(j):
    return jnp.full((16,), j, jnp.int32)


def _take16(vec, idx16):
    return lax.gather(
        vec, idx16[:, None],
        lax.GatherDimensionNumbers(offset_dims=(), collapsed_slice_dims=(0,),
                                   start_index_map=(0,)),
        slice_sizes=(1,), mode=lax.GatherScatterMode.PROMISE_IN_BOUNDS)


def _mesh():
    return plsc.VectorSubcoreMesh(
        core_axis_name="c", subcore_axis_name="s",
        num_cores=_NC, num_subcores=_NS)


def _zero_acc(sid, zbuf_v, acc_sh, zb):
    z16 = jnp.zeros((16,), jnp.float32)

    def zrow(i, c):
        zbuf_v[i, pl.ds(0, 16)] = z16
        zbuf_v[i, pl.ds(16, 16)] = z16
        return c
    lax.fori_loop(0, zb, zrow, 0)
    row0 = sid * _R0

    def zcp(i, c):
        pltpu.sync_copy(zbuf_v, acc_sh.at[pl.ds(row0 + i * zb, zb)])
        return c
    lax.fori_loop(0, _ZT // zb, zcp, 0)
    tail = row0 + _ZT

    @pl.when(sid < 15)
    def _():
        pltpu.sync_copy(zbuf_v.at[pl.ds(0, _R0 - _ZT)],
                        acc_sh.at[pl.ds(tail, _R0 - _ZT)])

    @pl.when(sid == 15)
    def _():
        pltpu.sync_copy(zbuf_v.at[pl.ds(0, _R15 - _ZT)],
                        acc_sh.at[pl.ds(tail, _R15 - _ZT)])
    return row0


def _readout(cid, sid, row0, acc_sh, out_hbm):
    @pl.when(sid < 15)
    def _():
        pltpu.sync_copy(acc_sh.at[pl.ds(row0, _R0)],
                        out_hbm.at[pl.ds(cid * _N + row0, _R0)])

    @pl.when(sid == 15)
    def _():
        pltpu.sync_copy(acc_sh.at[pl.ds(row0, _R15)],
                        out_hbm.at[pl.ds(cid * _N + row0, _R15)])


def _spmm_body(x_hbm, src_hbm, dst_hbm, attr_hbm, out_hbm,
               src_v, dst_v, attr_v, rows_v, zbuf_v, acc_sh, sem):
    cid = lax.axis_index("c")
    sid = lax.axis_index("s")
    row0 = _zero_acc(sid, zbuf_v, acc_sh, 128)
    plsc.subcore_barrier()

    ebase = sid * _EPT
    off = cid * _N

    def chunk(k, carry):
        c0 = ebase + k * _CS
        pltpu.sync_copy(src_hbm.at[pl.ds(c0, _CS)], src_v)
        pltpu.sync_copy(dst_hbm.at[pl.ds(c0, _CS)], dst_v)
        pltpu.sync_copy(attr_hbm.at[pl.ds(c0, _CS)], attr_v)

        def addoff(j, c):
            src_v[pl.ds(j * 16, 16)] = src_v[pl.ds(j * 16, 16)] + off
            return c
        lax.fori_loop(0, _CS // 16, addoff, 0)
        pltpu.async_copy(x_hbm.at[src_v], rows_v, sem).wait()

        def scale(g, c):
            a16 = attr_v[pl.ds(g * 16, 16)]
            for j in range(16):
                e = g * 16 + j
                a = _take16(a16, _lane(j))
                rows_v[e, pl.ds(0, 16)] = rows_v[e, pl.ds(0, 16)] * a
                rows_v[e, pl.ds(16, 16)] = rows_v[e, pl.ds(16, 16)] * a
            return c
        lax.fori_loop(0, _CS // 16, scale, 0)
        pltpu.sync_copy(rows_v, acc_sh.at[dst_v], add=True)
        return carry
    lax.fori_loop(0, _EPT // _CS, chunk, 0)
    plsc.subcore_barrier()
    _readout(cid, sid, row0, acc_sh, out_hbm)


def _sc_spmm(x2, src, dst, attr):
    f = pl.kernel(
        _spmm_body,
        out_type=jax.ShapeDtypeStruct((_NC * _N, _H), jnp.float32),
        mesh=_mesh(),
        compiler_params=pltpu.CompilerParams(use_tc_tiling_on_sc=False),
        scratch_types=[
            pltpu.VMEM((_CS,), jnp.int32),
            pltpu.VMEM((_CS,), jnp.int32),
            pltpu.VMEM((_CS,), jnp.float32),
            pltpu.VMEM((_CS, _H), jnp.float32),
            pltpu.VMEM((128, _H), jnp.float32),
            pltpu.VMEM_SHARED((_N, _H), jnp.float32),
            pltpu.SemaphoreType.DMA,
        ],
    )
    return f(x2, src, dst, attr)


def _bgc_edge_body(hl_hbm, hr_hbm, src_hbm, dst_hbm, ef_hbm, w_hbm, out_hbm,
                   srco_v, dsto_v, ef_v, rowsa_v, rowsb_v, w_v, sema, semb):
    cid = lax.axis_index("c")
    sid = lax.axis_index("s")
    pltpu.sync_copy(w_hbm.at[cid], w_v)

    w0 = w_v[pl.ds(0, 16)]
    w1 = w_v[pl.ds(16, 16)]
    ebase = sid * _EPT
    off = cid * _N

    def chunk(k, carry):
        c0 = ebase + k * _CB
        pltpu.sync_copy(src_hbm.at[pl.ds(c0, _CB)], srco_v)
        pltpu.sync_copy(dst_hbm.at[pl.ds(c0, _CB)], dsto_v)
        pltpu.sync_copy(ef_hbm.at[pl.ds(c0, _CB)], ef_v)

        def addoff(j, c):
            sl = pl.ds(j * 16, 16)
            srco_v[sl] = srco_v[sl] + off
            dsto_v[sl] = dsto_v[sl] + off
            return c
        lax.fori_loop(0, _CB // 16, addoff, 0)
        cpa = pltpu.async_copy(hl_hbm.at[dsto_v], rowsa_v, sema)
        cpb = pltpu.async_copy(hr_hbm.at[srco_v], rowsb_v, semb)
        cpa.wait()
        cpb.wait()

        def combine(g, c):
            a16 = ef_v[pl.ds(g * 16, 16)]
            for j in range(16):
                e = g * 16 + j
                a = _take16(a16, _lane(j))
                s0 = pl.ds(0, 16)
                s1 = pl.ds(16, 16)
                p0 = (rowsa_v[e, s0] + a * w0) + rowsb_v[e, s0]
                p1 = (rowsa_v[e, s1] + a * w1) + rowsb_v[e, s1]
                rowsa_v[e, s0] = jnp.maximum(p0, 0.0)
                rowsa_v[e, s1] = jnp.maximum(p1, 0.0)
            return c
        lax.fori_loop(0, _CB // 16, combine, 0)
        pltpu.sync_copy(rowsa_v, out_hbm.at[pl.ds(cid * _E + c0, _CB)])
        return carry
    lax.fori_loop(0, _EPT // _CB, chunk, 0)


def _sc_bgc_edge(hl2, hr2, src, dst, ef, w2):
    f = pl.kernel(
        _bgc_edge_body,
        out_type=jax.ShapeDtypeStruct((_NC * _E, _H), jnp.float32),
        mesh=_mesh(),
        compiler_params=pltpu.CompilerParams(use_tc_tiling_on_sc=False),
        scratch_types=[
            pltpu.VMEM((_CB,), jnp.int32),
            pltpu.VMEM((_CB,), jnp.int32),
            pltpu.VMEM((_CB,), jnp.float32),
            pltpu.VMEM((_CB, _H), jnp.float32),
            pltpu.VMEM((_CB, _H), jnp.float32),
            pltpu.VMEM((_H,), jnp.float32),
            pltpu.SemaphoreType.DMA,
            pltpu.SemaphoreType.DMA,
        ],
    )
    return f(hl2, hr2, src, dst, ef, w2)


def _scatter_body(msg_hbm, dst_hbm, out_hbm, dst_v, rows_v, zbuf_v, acc_sh):
    cid = lax.axis_index("c")
    sid = lax.axis_index("s")
    row0 = _zero_acc(sid, zbuf_v, acc_sh, 128)
    plsc.subcore_barrier()

    ebase = sid * _EPT

    def chunk(k, carry):
        c0 = ebase + k * _CS
        pltpu.sync_copy(dst_hbm.at[pl.ds(c0, _CS)], dst_v)
        pltpu.sync_copy(msg_hbm.at[pl.ds(cid * _E + c0, _CS)], rows_v)
        pltpu.sync_copy(rows_v, acc_sh.at[dst_v], add=True)
        return carry
    lax.fori_loop(0, _EPT // _CS, chunk, 0)
    plsc.subcore_barrier()
    _readout(cid, sid, row0, acc_sh, out_hbm)


def _sc_scatter(msg2, dst):
    f = pl.kernel(
        _scatter_body,
        out_type=jax.ShapeDtypeStruct((_NC * _N, _H), jnp.float32),
        mesh=_mesh(),
        compiler_params=pltpu.CompilerParams(use_tc_tiling_on_sc=False),
        scratch_types=[
            pltpu.VMEM((_CS,), jnp.int32),
            pltpu.VMEM((_CS, _H), jnp.float32),
            pltpu.VMEM((128, _H), jnp.float32),
            pltpu.VMEM_SHARED((_N, _H), jnp.float32),
        ],
    )
    return f(msg2, dst)


def _full(shape):
    return pl.BlockSpec(shape, lambda i: tuple(0 for _ in shape))


def _tc_mlp2(x, W1, b1, W2, b2):
    F = x.shape[1]

    def body(x_ref, w1_ref, b1_ref, w2_ref, b2_ref, o_ref):
        h = jnp.maximum(x_ref[...] @ w1_ref[...] + b1_ref[...], 0.0)
        o_ref[...] = jnp.maximum(h @ w2_ref[...] + b2_ref[...], 0.0)

    return pl.pallas_call(
        body,
        grid=(_NBLK,),
        in_specs=[pl.BlockSpec((_B, F), lambda i: (i, 0)),
                  _full((F, _EMB)), _full((1, _EMB)),
                  _full((_EMB, _EMB)), _full((1, _EMB))],
        out_specs=pl.BlockSpec((_B, _EMB), lambda i: (i, 0)),
        out_shape=jax.ShapeDtypeStruct((_N, _EMB), jnp.float32),
    )(x, W1, b1.reshape(1, -1), W2, b2.reshape(1, -1))


def _tc_mlp2_stacked(x, W1, b1, W2, b2):
    def body(x_ref, w1_ref, b1_ref, w2_ref, b2_ref, o_ref):
        h = jnp.maximum(x_ref[...] @ w1_ref[...] + b1_ref[...], 0.0)
        h2 = jnp.maximum(h @ w2_ref[...] + b2_ref[...], 0.0)
        o_ref[0] = h2[:, :_H]
        o_ref[1] = h2[:, _H:]

    return pl.pallas_call(
        body,
        grid=(_NBLK,),
        in_specs=[pl.BlockSpec((_B, _EMB), lambda i: (i, 0)),
                  _full((_EMB, _EMB)), _full((1, _EMB)),
                  _full((_EMB, _EMB)), _full((1, _EMB))],
        out_specs=pl.BlockSpec((2, _B, _H), lambda i: (0, i, 0)),
        out_shape=jax.ShapeDtypeStruct((2, _N, _H), jnp.float32),
    )(x, W1, b1.reshape(1, -1), W2, b2.reshape(1, -1))


def _tc_bgc_pre(right, left, fml_W, fml_b, fmr_W):
    def body(r_ref, l_ref, wl_ref, bl_ref, wr_ref, hl_ref, hr_ref):
        hl = r_ref[...] @ wl_ref[...] + bl_ref[...]
        hr = l_ref[...] @ wr_ref[...]
        hl_ref[0] = hl[:, :_H]
        hl_ref[1] = hl[:, _H:]
        hr_ref[0] = hr[:, :_H]
        hr_ref[1] = hr[:, _H:]

    return pl.pallas_call(
        body,
        grid=(_NBLK,),
        in_specs=[pl.BlockSpec((_B, _EMB), lambda i: (i, 0)),
                  pl.BlockSpec((_B, _EMB), lambda i: (i, 0)),
                  _full((_EMB, _EMB)), _full((1, _EMB)), _full((_EMB, _EMB))],
        out_specs=[pl.BlockSpec((2, _B, _H), lambda i: (0, i, 0)),
                   pl.BlockSpec((2, _B, _H), lambda i: (0, i, 0))],
        out_shape=[jax.ShapeDtypeStruct((2, _N, _H), jnp.float32),
                   jax.ShapeDtypeStruct((2, _N, _H), jnp.float32)],
    )(right, left, fml_W, fml_b.reshape(1, -1), fmr_W)


def _agg_specs():
    return [pl.BlockSpec((_B, _H), lambda i: (i, 0)),
            pl.BlockSpec((_B, _H), lambda i: (i + _NBLK, 0))]


def _tc_msg(pre2, fmf_W, fmf_b):
    def body(xl_ref, xh_ref, w_ref, b_ref, o_ref):
        x = jnp.concatenate([xl_ref[...], xh_ref[...]], axis=1)
        m = x @ w_ref[...] + b_ref[...]
        o_ref[0] = m[:, :_H]
        o_ref[1] = m[:, _H:]

    return pl.pallas_call(
        body,
        grid=(_EBLK,),
        in_specs=[pl.BlockSpec((_B, _H), lambda i: (i, 0)),
                  pl.BlockSpec((_B, _H), lambda i: (i + _EBLK, 0)),
                  _full((_EMB, _EMB)), _full((1, _EMB))],
        out_specs=pl.BlockSpec((2, _B, _H), lambda i: (0, i, 0)),
        out_shape=jax.ShapeDtypeStruct((2, _E, _H), jnp.float32),
    )(pre2, pre2, fmf_W, fmf_b.reshape(1, -1))


def _tc_bgc_post(agg, right, o1Wa, o1Wb, o1_b, o2_W, o2_b):
    def body(al_ref, ah_ref, r_ref, w1a_ref, w1b_ref, b1_ref,
             w2_ref, b2_ref, o_ref):
        a = jnp.concatenate([al_ref[...], ah_ref[...]], axis=1)
        h = jnp.maximum(a @ w1a_ref[...] + r_ref[...] @ w1b_ref[...]
                        + b1_ref[...], 0.0)
        o_ref[...] = h @ w2_ref[...] + b2_ref[...]

    return pl.pallas_call(
        body,
        grid=(_NBLK,),
        in_specs=_agg_specs() + [
            pl.BlockSpec((_B, _EMB), lambda i: (i, 0)),
            _full((_EMB, _EMB)), _full((_EMB, _EMB)),
            _full((1, _EMB)), _full((_EMB, _EMB)), _full((1, _EMB))],
        out_specs=pl.BlockSpec((_B, _EMB), lambda i: (i, 0)),
        out_shape=jax.ShapeDtypeStruct((_N, _EMB), jnp.float32),
    )(agg, agg, right, o1Wa, o1Wb, o1_b.reshape(1, -1), o2_W,
      o2_b.reshape(1, -1))


def _tc_layer_post(agg, z, zt, g, beta, jWa, jWb, jb):
    def body(al_ref, ah_ref, z_ref, zt_ref, g_ref, be_ref, ja_ref, jb_ref,
             bb_ref, nz_ref, zt2_ref):
        gc = jnp.concatenate([al_ref[...], ah_ref[...]], axis=1) + z_ref[...]
        m = jnp.mean(gc, axis=1, keepdims=True)
        d = gc - m
        v = jnp.mean(d * d, axis=1, keepdims=True)
        nz = d / jnp.sqrt(v + 1e-5) * g_ref[...] + be_ref[...]
        nz_ref[...] = nz
        u = nz @ ja_ref[...] + zt_ref[...] @ jb_ref[...] + bb_ref[...]
        zt2_ref[...] = 1.0 / (1.0 + jnp.exp(-u))

    return pl.pallas_call(
        body,
        grid=(_NBLK,),
        in_specs=_agg_specs() + [
            pl.BlockSpec((_B, _EMB), lambda i: (i, 0)),
            pl.BlockSpec((_B, _EMB), lambda i: (i, 0)),
            _full((1, _EMB)), _full((1, _EMB)),
            _full((_EMB, _EMB)), _full((_EMB, _EMB)), _full((1, _EMB))],
        out_specs=[pl.BlockSpec((_B, _EMB), lambda i: (i, 0)),
                   pl.BlockSpec((_B, _EMB), lambda i: (i, 0))],
        out_shape=[jax.ShapeDtypeStruct((_N, _EMB), jnp.float32),
                   jax.ShapeDtypeStruct((_N, _EMB), jnp.float32)],
    )(agg, agg, z, zt, g.reshape(1, -1), beta.reshape(1, -1), jWa, jWb,
      jb.reshape(1, -1))


def _bgc(left, src, dst, ef, right, p):
    hl2, hr2 = _tc_bgc_pre(right, left, p["fml_W"], p["fml_b"], p["fmr_W"])
    pre2 = _sc_bgc_edge(hl2.reshape(_NC * _N, _H), hr2.reshape(_NC * _N, _H),
                        src, dst, ef, p["fme_W"].reshape(_NC, _H))
    msg2 = _tc_msg(pre2, p["fmf_W"], p["fmf_b"])
    agg = _sc_scatter(msg2.reshape(_NC * _E, _H), dst)
    return _tc_bgc_post(agg, right, p["o1_W"][:_EMB],
                        p["o1_W"][_EMB:], p["o1_b"], p["o2_W"], p["o2_b"])


def kernel(constraint_features, edge_indices, edge_attrs, variable_features,
           params):
    e0 = edge_indices[0]
    e1 = edge_indices[1]
    ef = edge_attrs[:, 0]
    P = params

    cons = _tc_mlp2(constraint_features, P["ceW1"], P["ceb1"], P["ceW2"],
                    P["ceb2"])
    var = _tc_mlp2(variable_features, P["veW1"], P["veb1"], P["veW2"],
                   P["veb2"])
    cons = _bgc(var, e1, e0, ef, cons, P["v2c"])
    var = _bgc(cons, e0, e1, ef, var, P["c2v"])

    zc, zv, ztc, ztv = cons, var, cons, var
    for p in P["layers"]:
        mc2 = _tc_mlp2_stacked(zc, p["cW1"], p["cb1"], p["cW2"], p["cb2"])
        mv2 = _tc_mlp2_stacked(zv, p["vW1"], p["vb1"], p["vW2"], p["vb2"])
        aggc = _sc_spmm(mv2.reshape(_NC * _N, _H), e1, e0, ef)
        aggv = _sc_spmm(mc2.reshape(_NC * _N, _H), e0, e1, ef)
        zc, ztc = _tc_layer_post(aggc, zc, ztc, p["c_g"], p["c_b"],
                                 p["jcW"][:_EMB], p["jcW"][_EMB:], p["jcb"])
        zv, ztv = _tc_layer_post(aggv, zv, ztv, p["v_g"], p["v_b"],
                                 p["jvW"][:_EMB], p["jvW"][_EMB:], p["jvb"])
    return zc, zv, ztc, ztv


# speedup vs baseline: 5.5490x; 1.4565x over previous
import jax
import jax.numpy as jnp
from jax import lax
from jax.experimental import pallas as pl
from jax.experimental.pallas import tpu as pltpu
from jax.experimental.pallas import tpu_sc as plsc

_N = 50000
_E = 800000
_EMB = 64
_H = 32
_NC = 2
_NS = 16
_R0 = 3128
_R15 = _N - 15 * _R0
_ZT = 3072
_EPT = _E // _NS
_CS = 400
_CB = 400
_B = 2000
_NBLK = _N // _B
_EBLK = _E // _B


def _lane(j):
    return jnp.full((16,), j, jnp.int32)


def _take16(vec, idx16):
    return lax.gather(
        vec, idx16[:, None],
        lax.GatherDimensionNumbers(offset_dims=(), collapsed_slice_dims=(0,),
                                   start_index_map=(0,)),
        slice_sizes=(1,), mode=lax.GatherScatterMode.PROMISE_IN_BOUNDS)


def _mesh():
    return plsc.VectorSubcoreMesh(
        core_axis_name="c", subcore_axis_name="s",
        num_cores=_NC, num_subcores=_NS)


def _zero_acc(sid, zbuf_v, acc_sh, zb):
    z16 = jnp.zeros((16,), jnp.float32)

    def zrow(i, c):
        zbuf_v[i, pl.ds(0, 16)] = z16
        zbuf_v[i, pl.ds(16, 16)] = z16
        return c
    lax.fori_loop(0, zb, zrow, 0)
    row0 = sid * _R0

    def zcp(i, c):
        pltpu.sync_copy(zbuf_v, acc_sh.at[pl.ds(row0 + i * zb, zb)])
        return c
    lax.fori_loop(0, _ZT // zb, zcp, 0)
    tail = row0 + _ZT

    @pl.when(sid < 15)
    def _():
        pltpu.sync_copy(zbuf_v.at[pl.ds(0, _R0 - _ZT)],
                        acc_sh.at[pl.ds(tail, _R0 - _ZT)])

    @pl.when(sid == 15)
    def _():
        pltpu.sync_copy(zbuf_v.at[pl.ds(0, _R15 - _ZT)],
                        acc_sh.at[pl.ds(tail, _R15 - _ZT)])
    return row0


def _readout(cid, sid, row0, acc_sh, out_hbm):
    @pl.when(sid < 15)
    def _():
        pltpu.sync_copy(acc_sh.at[pl.ds(row0, _R0)],
                        out_hbm.at[pl.ds(cid * _N + row0, _R0)])

    @pl.when(sid == 15)
    def _():
        pltpu.sync_copy(acc_sh.at[pl.ds(row0, _R15)],
                        out_hbm.at[pl.ds(cid * _N + row0, _R15)])


def _spmm_body(x_hbm, src_hbm, dst_hbm, attr_hbm, out_hbm,
               src0, dst0, attr0, rows0, src1, dst1, attr1, rows1,
               zbuf_v, acc_sh, gsem0, gsem1, isem):
    cid = lax.axis_index("c")
    sid = lax.axis_index("s")
    row0 = _zero_acc(sid, zbuf_v, acc_sh, 64)
    plsc.subcore_barrier()

    ebase = sid * _EPT
    off = cid * _N

    def load_idx(k, sv, dv, av):
        c0 = ebase + k * _CS
        pltpu.async_copy(src_hbm.at[pl.ds(c0, _CS)], sv, isem)
        pltpu.async_copy(dst_hbm.at[pl.ds(c0, _CS)], dv, isem)
        ca = pltpu.async_copy(attr_hbm.at[pl.ds(c0, _CS)], av, isem)
        pltpu.make_async_copy(src_hbm.at[pl.ds(c0, _CS)], sv, isem).wait()
        pltpu.make_async_copy(dst_hbm.at[pl.ds(c0, _CS)], dv, isem).wait()
        ca.wait()

        def lo(j, c):
            sv[pl.ds(j * 16, 16)] = sv[pl.ds(j * 16, 16)] + off
            return c
        lax.fori_loop(0, _CS // 16, lo, 0)

    def consume(av, rv, dv):
        def scale(g, c):
            a16 = av[pl.ds(g * 16, 16)]
            for j in range(16):
                e = g * 16 + j
                a = _take16(a16, _lane(j))
                rv[e, pl.ds(0, 16)] = rv[e, pl.ds(0, 16)] * a
                rv[e, pl.ds(16, 16)] = rv[e, pl.ds(16, 16)] * a
            return c
        lax.fori_loop(0, _CS // 16, scale, 0)
        pltpu.sync_copy(rv, acc_sh.at[dv], add=True)

    load_idx(0, src0, dst0, attr0)
    pltpu.async_copy(x_hbm.at[src0], rows0, gsem0)

    def body(g, carry):
        ka = 2 * g
        load_idx(ka + 1, src1, dst1, attr1)
        pltpu.async_copy(x_hbm.at[src1], rows1, gsem1)
        pltpu.make_async_copy(x_hbm.at[src0], rows0, gsem0).wait()
        consume(attr0, rows0, dst0)
        load_idx(ka + 2, src0, dst0, attr0)
        pltpu.async_copy(x_hbm.at[src0], rows0, gsem0)
        pltpu.make_async_copy(x_hbm.at[src1], rows1, gsem1).wait()
        consume(attr1, rows1, dst1)
        return carry
    lax.fori_loop(0, (_EPT // _CS - 1) // 2, body, 0)
    pltpu.make_async_copy(x_hbm.at[src0], rows0, gsem0).wait()
    consume(attr0, rows0, dst0)
    plsc.subcore_barrier()
    _readout(cid, sid, row0, acc_sh, out_hbm)


def _sc_spmm(x2, src, dst, attr):
    f = pl.kernel(
        _spmm_body,
        out_type=jax.ShapeDtypeStruct((_NC * _N, _H), jnp.float32),
        mesh=_mesh(),
        compiler_params=pltpu.CompilerParams(use_tc_tiling_on_sc=False),
        scratch_types=[
            pltpu.VMEM((_CS,), jnp.int32),
            pltpu.VMEM((_CS,), jnp.int32),
            pltpu.VMEM((_CS,), jnp.float32),
            pltpu.VMEM((_CS, _H), jnp.float32),
            pltpu.VMEM((_CS,), jnp.int32),
            pltpu.VMEM((_CS,), jnp.int32),
            pltpu.VMEM((_CS,), jnp.float32),
            pltpu.VMEM((_CS, _H), jnp.float32),
            pltpu.VMEM((64, _H), jnp.float32),
            pltpu.VMEM_SHARED((_N, _H), jnp.float32),
            pltpu.SemaphoreType.DMA,
            pltpu.SemaphoreType.DMA,
            pltpu.SemaphoreType.DMA,
        ],
    )
    return f(x2, src, dst, attr)


def _bgc_edge_body(hl_hbm, hr_hbm, src_hbm, dst_hbm, ef_hbm, w_hbm, out_hbm,
                   srco0, dsto0, ef0, rowsa0, rowsb0,
                   srco1, dsto1, ef1, rowsa1, rowsb1, w_v,
                   ga0, gb0, ga1, gb1, isem):
    cid = lax.axis_index("c")
    sid = lax.axis_index("s")
    pltpu.sync_copy(w_hbm.at[cid], w_v)

    w0 = w_v[pl.ds(0, 16)]
    w1 = w_v[pl.ds(16, 16)]
    ebase = sid * _EPT
    off = cid * _N

    def load_idx(k, sv, dv, ev):
        c0 = ebase + k * _CB
        pltpu.async_copy(src_hbm.at[pl.ds(c0, _CB)], sv, isem)
        pltpu.async_copy(dst_hbm.at[pl.ds(c0, _CB)], dv, isem)
        ca = pltpu.async_copy(ef_hbm.at[pl.ds(c0, _CB)], ev, isem)
        pltpu.make_async_copy(src_hbm.at[pl.ds(c0, _CB)], sv, isem).wait()
        pltpu.make_async_copy(dst_hbm.at[pl.ds(c0, _CB)], dv, isem).wait()
        ca.wait()

        def lo(j, c):
            sl = pl.ds(j * 16, 16)
            sv[sl] = sv[sl] + off
            dv[sl] = dv[sl] + off
            return c
        lax.fori_loop(0, _CB // 16, lo, 0)

    def start(dv, sv, ra, rb, sa, sb):
        pltpu.async_copy(hl_hbm.at[dv], ra, sa)
        pltpu.async_copy(hr_hbm.at[sv], rb, sb)

    def consume(k, dv, sv, ev, ra, rb, sa, sb):
        pltpu.make_async_copy(hl_hbm.at[dv], ra, sa).wait()
        pltpu.make_async_copy(hr_hbm.at[sv], rb, sb).wait()

        def combine(g, c):
            a16 = ev[pl.ds(g * 16, 16)]
            for j in range(16):
                e = g * 16 + j
                a = _take16(a16, _lane(j))
                s0 = pl.ds(0, 16)
                s1 = pl.ds(16, 16)
                p0 = (ra[e, s0] + a * w0) + rb[e, s0]
                p1 = (ra[e, s1] + a * w1) + rb[e, s1]
                ra[e, s0] = jnp.maximum(p0, 0.0)
                ra[e, s1] = jnp.maximum(p1, 0.0)
            return c
        lax.fori_loop(0, _CB // 16, combine, 0)
        pltpu.sync_copy(ra, out_hbm.at[pl.ds(cid * _E + ebase + k * _CB, _CB)])

    load_idx(0, srco0, dsto0, ef0)
    start(dsto0, srco0, rowsa0, rowsb0, ga0, gb0)

    def body(g, carry):
        ka = 2 * g
        load_idx(ka + 1, srco1, dsto1, ef1)
        start(dsto1, srco1, rowsa1, rowsb1, ga1, gb1)
        consume(ka, dsto0, srco0, ef0, rowsa0, rowsb0, ga0, gb0)
        load_idx(ka + 2, srco0, dsto0, ef0)
        start(dsto0, srco0, rowsa0, rowsb0, ga0, gb0)
        consume(ka + 1, dsto1, srco1, ef1, rowsa1, rowsb1, ga1, gb1)
        return carry
    lax.fori_loop(0, (_EPT // _CB - 1) // 2, body, 0)
    consume(_EPT // _CB - 1, dsto0, srco0, ef0, rowsa0, rowsb0, ga0, gb0)


def _sc_bgc_edge(hl2, hr2, src, dst, ef, w2):
    f = pl.kernel(
        _bgc_edge_body,
        out_type=jax.ShapeDtypeStruct((_NC * _E, _H), jnp.float32),
        mesh=_mesh(),
        compiler_params=pltpu.CompilerParams(use_tc_tiling_on_sc=False),
        scratch_types=[
            pltpu.VMEM((_CB,), jnp.int32),
            pltpu.VMEM((_CB,), jnp.int32),
            pltpu.VMEM((_CB,), jnp.float32),
            pltpu.VMEM((_CB, _H), jnp.float32),
            pltpu.VMEM((_CB, _H), jnp.float32),
            pltpu.VMEM((_CB,), jnp.int32),
            pltpu.VMEM((_CB,), jnp.int32),
            pltpu.VMEM((_CB,), jnp.float32),
            pltpu.VMEM((_CB, _H), jnp.float32),
            pltpu.VMEM((_CB, _H), jnp.float32),
            pltpu.VMEM((_H,), jnp.float32),
            pltpu.SemaphoreType.DMA,
            pltpu.SemaphoreType.DMA,
            pltpu.SemaphoreType.DMA,
            pltpu.SemaphoreType.DMA,
            pltpu.SemaphoreType.DMA,
        ],
    )
    return f(hl2, hr2, src, dst, ef, w2)


def _scatter_body(msg_hbm, dst_hbm, out_hbm, dst0, rows0, dst1, rows1,
                  zbuf_v, acc_sh, m0, m1, isem):
    cid = lax.axis_index("c")
    sid = lax.axis_index("s")
    row0 = _zero_acc(sid, zbuf_v, acc_sh, 64)
    plsc.subcore_barrier()

    ebase = sid * _EPT

    def load(k, dv, rv, ms):
        c0 = ebase + k * _CS
        ca = pltpu.async_copy(dst_hbm.at[pl.ds(c0, _CS)], dv, isem)
        pltpu.async_copy(msg_hbm.at[pl.ds(cid * _E + c0, _CS)], rv, ms)
        ca.wait()

    def consume(k, dv, rv, ms):
        c0 = ebase + k * _CS
        pltpu.make_async_copy(msg_hbm.at[pl.ds(cid * _E + c0, _CS)],
                              rv, ms).wait()
        pltpu.sync_copy(rv, acc_sh.at[dv], add=True)

    load(0, dst0, rows0, m0)

    def body(g, carry):
        ka = 2 * g
        load(ka + 1, dst1, rows1, m1)
        consume(ka, dst0, rows0, m0)
        load(ka + 2, dst0, rows0, m0)
        consume(ka + 1, dst1, rows1, m1)
        return carry
    lax.fori_loop(0, (_EPT // _CS - 1) // 2, body, 0)
    consume(_EPT // _CS - 1, dst0, rows0, m0)
    plsc.subcore_barrier()
    _readout(cid, sid, row0, acc_sh, out_hbm)


def _sc_scatter(msg2, dst):
    f = pl.kernel(
        _scatter_body,
        out_type=jax.ShapeDtypeStruct((_NC * _N, _H), jnp.float32),
        mesh=_mesh(),
        compiler_params=pltpu.CompilerParams(use_tc_tiling_on_sc=False),
        scratch_types=[
            pltpu.VMEM((_CS,), jnp.int32),
            pltpu.VMEM((_CS, _H), jnp.float32),
            pltpu.VMEM((_CS,), jnp.int32),
            pltpu.VMEM((_CS, _H), jnp.float32),
            pltpu.VMEM((64, _H), jnp.float32),
            pltpu.VMEM_SHARED((_N, _H), jnp.float32),
            pltpu.SemaphoreType.DMA,
            pltpu.SemaphoreType.DMA,
            pltpu.SemaphoreType.DMA,
        ],
    )
    return f(msg2, dst)


def _full(shape):
    return pl.BlockSpec(shape, lambda i: tuple(0 for _ in shape))


def _tc_mlp2(x, W1, b1, W2, b2):
    F = x.shape[1]

    def body(x_ref, w1_ref, b1_ref, w2_ref, b2_ref, o_ref):
        h = jnp.maximum(x_ref[...] @ w1_ref[...] + b1_ref[...], 0.0)
        o_ref[...] = jnp.maximum(h @ w2_ref[...] + b2_ref[...], 0.0)

    return pl.pallas_call(
        body,
        grid=(_NBLK,),
        in_specs=[pl.BlockSpec((_B, F), lambda i: (i, 0)),
                  _full((F, _EMB)), _full((1, _EMB)),
                  _full((_EMB, _EMB)), _full((1, _EMB))],
        out_specs=pl.BlockSpec((_B, _EMB), lambda i: (i, 0)),
        out_shape=jax.ShapeDtypeStruct((_N, _EMB), jnp.float32),
    )(x, W1, b1.reshape(1, -1), W2, b2.reshape(1, -1))


def _tc_mlp2_stacked(x, W1, b1, W2, b2):
    def body(x_ref, w1_ref, b1_ref, w2_ref, b2_ref, o_ref):
        h = jnp.maximum(x_ref[...] @ w1_ref[...] + b1_ref[...], 0.0)
        h2 = jnp.maximum(h @ w2_ref[...] + b2_ref[...], 0.0)
        o_ref[0] = h2[:, :_H]
        o_ref[1] = h2[:, _H:]

    return pl.pallas_call(
        body,
        grid=(_NBLK,),
        in_specs=[pl.BlockSpec((_B, _EMB), lambda i: (i, 0)),
                  _full((_EMB, _EMB)), _full((1, _EMB)),
                  _full((_EMB, _EMB)), _full((1, _EMB))],
        out_specs=pl.BlockSpec((2, _B, _H), lambda i: (0, i, 0)),
        out_shape=jax.ShapeDtypeStruct((2, _N, _H), jnp.float32),
    )(x, W1, b1.reshape(1, -1), W2, b2.reshape(1, -1))


def _tc_bgc_pre(right, left, fml_W, fml_b, fmr_W):
    def body(r_ref, l_ref, wl_ref, bl_ref, wr_ref, hl_ref, hr_ref):
        hl = r_ref[...] @ wl_ref[...] + bl_ref[...]
        hr = l_ref[...] @ wr_ref[...]
        hl_ref[0] = hl[:, :_H]
        hl_ref[1] = hl[:, _H:]
        hr_ref[0] = hr[:, :_H]
        hr_ref[1] = hr[:, _H:]

    return pl.pallas_call(
        body,
        grid=(_NBLK,),
        in_specs=[pl.BlockSpec((_B, _EMB), lambda i: (i, 0)),
                  pl.BlockSpec((_B, _EMB), lambda i: (i, 0)),
                  _full((_EMB, _EMB)), _full((1, _EMB)), _full((_EMB, _EMB))],
        out_specs=[pl.BlockSpec((2, _B, _H), lambda i: (0, i, 0)),
                   pl.BlockSpec((2, _B, _H), lambda i: (0, i, 0))],
        out_shape=[jax.ShapeDtypeStruct((2, _N, _H), jnp.float32),
                   jax.ShapeDtypeStruct((2, _N, _H), jnp.float32)],
    )(right, left, fml_W, fml_b.reshape(1, -1), fmr_W)


def _agg_specs():
    return [pl.BlockSpec((_B, _H), lambda i: (i, 0)),
            pl.BlockSpec((_B, _H), lambda i: (i + _NBLK, 0))]


def _tc_msg(pre2, fmf_W, fmf_b):
    def body(xl_ref, xh_ref, w_ref, b_ref, o_ref):
        x = jnp.concatenate([xl_ref[...], xh_ref[...]], axis=1)
        m = x @ w_ref[...] + b_ref[...]
        o_ref[0] = m[:, :_H]
        o_ref[1] = m[:, _H:]

    return pl.pallas_call(
        body,
        grid=(_EBLK,),
        in_specs=[pl.BlockSpec((_B, _H), lambda i: (i, 0)),
                  pl.BlockSpec((_B, _H), lambda i: (i + _EBLK, 0)),
                  _full((_EMB, _EMB)), _full((1, _EMB))],
        out_specs=pl.BlockSpec((2, _B, _H), lambda i: (0, i, 0)),
        out_shape=jax.ShapeDtypeStruct((2, _E, _H), jnp.float32),
    )(pre2, pre2, fmf_W, fmf_b.reshape(1, -1))


def _tc_bgc_post(agg, right, o1Wa, o1Wb, o1_b, o2_W, o2_b):
    def body(al_ref, ah_ref, r_ref, w1a_ref, w1b_ref, b1_ref,
             w2_ref, b2_ref, o_ref):
        a = jnp.concatenate([al_ref[...], ah_ref[...]], axis=1)
        h = jnp.maximum(a @ w1a_ref[...] + r_ref[...] @ w1b_ref[...]
                        + b1_ref[...], 0.0)
        o_ref[...] = h @ w2_ref[...] + b2_ref[...]

    return pl.pallas_call(
        body,
        grid=(_NBLK,),
        in_specs=_agg_specs() + [
            pl.BlockSpec((_B, _EMB), lambda i: (i, 0)),
            _full((_EMB, _EMB)), _full((_EMB, _EMB)),
            _full((1, _EMB)), _full((_EMB, _EMB)), _full((1, _EMB))],
        out_specs=pl.BlockSpec((_B, _EMB), lambda i: (i, 0)),
        out_shape=jax.ShapeDtypeStruct((_N, _EMB), jnp.float32),
    )(agg, agg, right, o1Wa, o1Wb, o1_b.reshape(1, -1), o2_W,
      o2_b.reshape(1, -1))


def _tc_layer_post(agg, z, zt, g, beta, jWa, jWb, jb):
    def body(al_ref, ah_ref, z_ref, zt_ref, g_ref, be_ref, ja_ref, jb_ref,
             bb_ref, nz_ref, zt2_ref):
        gc = jnp.concatenate([al_ref[...], ah_ref[...]], axis=1) + z_ref[...]
        m = jnp.mean(gc, axis=1, keepdims=True)
        d = gc - m
        v = jnp.mean(d * d, axis=1, keepdims=True)
        nz = d / jnp.sqrt(v + 1e-5) * g_ref[...] + be_ref[...]
        nz_ref[...] = nz
        u = nz @ ja_ref[...] + zt_ref[...] @ jb_ref[...] + bb_ref[...]
        zt2_ref[...] = 1.0 / (1.0 + jnp.exp(-u))

    return pl.pallas_call(
        body,
        grid=(_NBLK,),
        in_specs=_agg_specs() + [
            pl.BlockSpec((_B, _EMB), lambda i: (i, 0)),
            pl.BlockSpec((_B, _EMB), lambda i: (i, 0)),
            _full((1, _EMB)), _full((1, _EMB)),
            _full((_EMB, _EMB)), _full((_EMB, _EMB)), _full((1, _EMB))],
        out_specs=[pl.BlockSpec((_B, _EMB), lambda i: (i, 0)),
                   pl.BlockSpec((_B, _EMB), lambda i: (i, 0))],
        out_shape=[jax.ShapeDtypeStruct((_N, _EMB), jnp.float32),
                   jax.ShapeDtypeStruct((_N, _EMB), jnp.float32)],
    )(agg, agg, z, zt, g.reshape(1, -1), beta.reshape(1, -1), jWa, jWb,
      jb.reshape(1, -1))


def _bgc(left, src, dst, ef, right, p):
    hl2, hr2 = _tc_bgc_pre(right, left, p["fml_W"], p["fml_b"], p["fmr_W"])
    pre2 = _sc_bgc_edge(hl2.reshape(_NC * _N, _H), hr2.reshape(_NC * _N, _H),
                        src, dst, ef, p["fme_W"].reshape(_NC, _H))
    msg2 = _tc_msg(pre2, p["fmf_W"], p["fmf_b"])
    agg = _sc_scatter(msg2.reshape(_NC * _E, _H), dst)
    return _tc_bgc_post(agg, right, p["o1_W"][:_EMB],
                        p["o1_W"][_EMB:], p["o1_b"], p["o2_W"], p["o2_b"])


def kernel(constraint_features, edge_indices, edge_attrs, variable_features,
           params):
    e0 = edge_indices[0]
    e1 = edge_indices[1]
    ef = edge_attrs[:, 0]
    P = params

    cons = _tc_mlp2(constraint_features, P["ceW1"], P["ceb1"], P["ceW2"],
                    P["ceb2"])
    var = _tc_mlp2(variable_features, P["veW1"], P["veb1"], P["veW2"],
                   P["veb2"])
    cons = _bgc(var, e1, e0, ef, cons, P["v2c"])
    var = _bgc(cons, e0, e1, ef, var, P["c2v"])

    zc, zv, ztc, ztv = cons, var, cons, var
    for p in P["layers"]:
        mc2 = _tc_mlp2_stacked(zc, p["cW1"], p["cb1"], p["cW2"], p["cb2"])
        mv2 = _tc_mlp2_stacked(zv, p["vW1"], p["vb1"], p["vW2"], p["vb2"])
        aggc = _sc_spmm(mv2.reshape(_NC * _N, _H), e1, e0, ef)
        aggv = _sc_spmm(mc2.reshape(_NC * _N, _H), e0, e1, ef)
        zc, ztc = _tc_layer_post(aggc, zc, ztc, p["c_g"], p["c_b"],
                                 p["jcW"][:_EMB], p["jcW"][_EMB:], p["jcb"])
        zv, ztv = _tc_layer_post(aggv, zv, ztv, p["v_g"], p["v_b"],
                                 p["jvW"][:_EMB], p["jvW"][_EMB:], p["jvb"])
    return zc, zv, ztc, ztv


# speedup vs baseline: 5.5513x; 1.0004x over previous
import jax
import jax.numpy as jnp
from jax import lax
from jax.experimental import pallas as pl
from jax.experimental.pallas import tpu as pltpu
from jax.experimental.pallas import tpu_sc as plsc

_N = 50000
_E = 800000
_EMB = 64
_H = 32
_NC = 2
_NS = 16
_R0 = 3128
_R15 = _N - 15 * _R0
_ZT = 3072
_EPT = _E // _NS
_CS = 400
_CB = 400
_B = 2000
_NBLK = _N // _B
_EBLK = _E // _B


def _lane(j):
    return jnp.full((16,), j, jnp.int32)


def _take16(vec, idx16):
    return lax.gather(
        vec, idx16[:, None],
        lax.GatherDimensionNumbers(offset_dims=(), collapsed_slice_dims=(0,),
                                   start_index_map=(0,)),
        slice_sizes=(1,), mode=lax.GatherScatterMode.PROMISE_IN_BOUNDS)


def _mesh():
    return plsc.VectorSubcoreMesh(
        core_axis_name="c", subcore_axis_name="s",
        num_cores=_NC, num_subcores=_NS)


def _zero_acc(sid, zbuf_v, acc_sh, zb):
    z16 = jnp.zeros((16,), jnp.float32)

    def zrow(i, c):
        zbuf_v[i, pl.ds(0, 16)] = z16
        zbuf_v[i, pl.ds(16, 16)] = z16
        return c
    lax.fori_loop(0, zb, zrow, 0)
    row0 = sid * _R0

    def zcp(i, c):
        pltpu.sync_copy(zbuf_v, acc_sh.at[pl.ds(row0 + i * zb, zb)])
        return c
    lax.fori_loop(0, _ZT // zb, zcp, 0)
    tail = row0 + _ZT

    @pl.when(sid < 15)
    def _():
        pltpu.sync_copy(zbuf_v.at[pl.ds(0, _R0 - _ZT)],
                        acc_sh.at[pl.ds(tail, _R0 - _ZT)])

    @pl.when(sid == 15)
    def _():
        pltpu.sync_copy(zbuf_v.at[pl.ds(0, _R15 - _ZT)],
                        acc_sh.at[pl.ds(tail, _R15 - _ZT)])
    return row0


def _readout(cid, sid, row0, acc_sh, out_hbm):
    @pl.when(sid < 15)
    def _():
        pltpu.sync_copy(acc_sh.at[pl.ds(row0, _R0)],
                        out_hbm.at[pl.ds(cid * _N + row0, _R0)])

    @pl.when(sid == 15)
    def _():
        pltpu.sync_copy(acc_sh.at[pl.ds(row0, _R15)],
                        out_hbm.at[pl.ds(cid * _N + row0, _R15)])


def _spmm_body(x_hbm, src_hbm, dst_hbm, attr_hbm, out_hbm,
               src0, dst0, attr0, rows0, src1, dst1, attr1, rows1,
               zbuf_v, acc_sh, gsem0, gsem1, isem):
    cid = lax.axis_index("c")
    sid = lax.axis_index("s")
    row0 = _zero_acc(sid, zbuf_v, acc_sh, 64)
    plsc.subcore_barrier()

    ebase = sid * _EPT
    off = cid * _N

    def load_idx(k, sv, dv, av):
        c0 = ebase + k * _CS
        pltpu.async_copy(src_hbm.at[pl.ds(c0, _CS)], sv, isem)
        pltpu.async_copy(dst_hbm.at[pl.ds(c0, _CS)], dv, isem)
        ca = pltpu.async_copy(attr_hbm.at[pl.ds(c0, _CS)], av, isem)
        pltpu.make_async_copy(src_hbm.at[pl.ds(c0, _CS)], sv, isem).wait()
        pltpu.make_async_copy(dst_hbm.at[pl.ds(c0, _CS)], dv, isem).wait()
        ca.wait()

        def lo(j, c):
            sv[pl.ds(j * 16, 16)] = sv[pl.ds(j * 16, 16)] + off
            return c
        lax.fori_loop(0, _CS // 16, lo, 0)

    def consume(av, rv, dv):
        def scale(g, c):
            a16 = av[pl.ds(g * 16, 16)]
            for j in range(16):
                e = g * 16 + j
                a = _take16(a16, _lane(j))
                rv[e, pl.ds(0, 16)] = rv[e, pl.ds(0, 16)] * a
                rv[e, pl.ds(16, 16)] = rv[e, pl.ds(16, 16)] * a
            return c
        lax.fori_loop(0, _CS // 16, scale, 0)
        pltpu.sync_copy(rv, acc_sh.at[dv], add=True)

    load_idx(0, src0, dst0, attr0)
    pltpu.async_copy(x_hbm.at[src0], rows0, gsem0)

    def body(g, carry):
        ka = 2 * g
        load_idx(ka + 1, src1, dst1, attr1)
        pltpu.async_copy(x_hbm.at[src1], rows1, gsem1)
        pltpu.make_async_copy(x_hbm.at[src0], rows0, gsem0).wait()
        consume(attr0, rows0, dst0)
        load_idx(ka + 2, src0, dst0, attr0)
        pltpu.async_copy(x_hbm.at[src0], rows0, gsem0)
        pltpu.make_async_copy(x_hbm.at[src1], rows1, gsem1).wait()
        consume(attr1, rows1, dst1)
        return carry
    lax.fori_loop(0, (_EPT // _CS - 1) // 2, body, 0)
    pltpu.make_async_copy(x_hbm.at[src0], rows0, gsem0).wait()
    consume(attr0, rows0, dst0)
    plsc.subcore_barrier()
    _readout(cid, sid, row0, acc_sh, out_hbm)


def _sc_spmm(x2, src, dst, attr):
    f = pl.kernel(
        _spmm_body,
        out_type=jax.ShapeDtypeStruct((_NC * _N, _H), jnp.float32),
        mesh=_mesh(),
        compiler_params=pltpu.CompilerParams(use_tc_tiling_on_sc=False),
        scratch_types=[
            pltpu.VMEM((_CS,), jnp.int32),
            pltpu.VMEM((_CS,), jnp.int32),
            pltpu.VMEM((_CS,), jnp.float32),
            pltpu.VMEM((_CS, _H), jnp.float32),
            pltpu.VMEM((_CS,), jnp.int32),
            pltpu.VMEM((_CS,), jnp.int32),
            pltpu.VMEM((_CS,), jnp.float32),
            pltpu.VMEM((_CS, _H), jnp.float32),
            pltpu.VMEM((64, _H), jnp.float32),
            pltpu.VMEM_SHARED((_N, _H), jnp.float32),
            pltpu.SemaphoreType.DMA,
            pltpu.SemaphoreType.DMA,
            pltpu.SemaphoreType.DMA,
        ],
    )
    return f(x2, src, dst, attr)


def _bgc_edge_body(hl_hbm, hr_hbm, src_hbm, dst_hbm, ef_hbm, w_hbm, out_hbm,
                   srco0, dsto0, ef0, rowsa0, rowsb0,
                   srco1, dsto1, ef1, rowsa1, rowsb1, w_v,
                   ga0, gb0, ga1, gb1, isem):
    cid = lax.axis_index("c")
    sid = lax.axis_index("s")
    pltpu.sync_copy(w_hbm.at[cid], w_v)

    w0 = w_v[pl.ds(0, 16)]
    w1 = w_v[pl.ds(16, 16)]
    ebase = sid * _EPT
    off = cid * _N

    def load_idx(k, sv, dv, ev):
        c0 = ebase + k * _CB
        pltpu.async_copy(src_hbm.at[pl.ds(c0, _CB)], sv, isem)
        pltpu.async_copy(dst_hbm.at[pl.ds(c0, _CB)], dv, isem)
        ca = pltpu.async_copy(ef_hbm.at[pl.ds(c0, _CB)], ev, isem)
        pltpu.make_async_copy(src_hbm.at[pl.ds(c0, _CB)], sv, isem).wait()
        pltpu.make_async_copy(dst_hbm.at[pl.ds(c0, _CB)], dv, isem).wait()
        ca.wait()

        def lo(j, c):
            sl = pl.ds(j * 16, 16)
            sv[sl] = sv[sl] + off
            dv[sl] = dv[sl] + off
            return c
        lax.fori_loop(0, _CB // 16, lo, 0)

    def start(dv, sv, ra, rb, sa, sb):
        pltpu.async_copy(hl_hbm.at[dv], ra, sa)
        pltpu.async_copy(hr_hbm.at[sv], rb, sb)

    def consume(k, dv, sv, ev, ra, rb, sa, sb):
        pltpu.make_async_copy(hl_hbm.at[dv], ra, sa).wait()
        pltpu.make_async_copy(hr_hbm.at[sv], rb, sb).wait()

        def combine(g, c):
            a16 = ev[pl.ds(g * 16, 16)]
            for j in range(16):
                e = g * 16 + j
                a = _take16(a16, _lane(j))
                s0 = pl.ds(0, 16)
                s1 = pl.ds(16, 16)
                p0 = (ra[e, s0] + a * w0) + rb[e, s0]
                p1 = (ra[e, s1] + a * w1) + rb[e, s1]
                ra[e, s0] = jnp.maximum(p0, 0.0)
                ra[e, s1] = jnp.maximum(p1, 0.0)
            return c
        lax.fori_loop(0, _CB // 16, combine, 0)
        pltpu.sync_copy(ra, out_hbm.at[pl.ds(cid * _E + ebase + k * _CB, _CB)])

    load_idx(0, srco0, dsto0, ef0)
    start(dsto0, srco0, rowsa0, rowsb0, ga0, gb0)

    def body(g, carry):
        ka = 2 * g
        load_idx(ka + 1, srco1, dsto1, ef1)
        start(dsto1, srco1, rowsa1, rowsb1, ga1, gb1)
        consume(ka, dsto0, srco0, ef0, rowsa0, rowsb0, ga0, gb0)
        load_idx(ka + 2, srco0, dsto0, ef0)
        start(dsto0, srco0, rowsa0, rowsb0, ga0, gb0)
        consume(ka + 1, dsto1, srco1, ef1, rowsa1, rowsb1, ga1, gb1)
        return carry
    lax.fori_loop(0, (_EPT // _CB - 1) // 2, body, 0)
    consume(_EPT // _CB - 1, dsto0, srco0, ef0, rowsa0, rowsb0, ga0, gb0)


def _sc_bgc_edge(hl2, hr2, src, dst, ef, w2):
    f = pl.kernel(
        _bgc_edge_body,
        out_type=jax.ShapeDtypeStruct((_NC * _E, _H), jnp.float32),
        mesh=_mesh(),
        compiler_params=pltpu.CompilerParams(use_tc_tiling_on_sc=False),
        scratch_types=[
            pltpu.VMEM((_CB,), jnp.int32),
            pltpu.VMEM((_CB,), jnp.int32),
            pltpu.VMEM((_CB,), jnp.float32),
            pltpu.VMEM((_CB, _H), jnp.float32),
            pltpu.VMEM((_CB, _H), jnp.float32),
            pltpu.VMEM((_CB,), jnp.int32),
            pltpu.VMEM((_CB,), jnp.int32),
            pltpu.VMEM((_CB,), jnp.float32),
            pltpu.VMEM((_CB, _H), jnp.float32),
            pltpu.VMEM((_CB, _H), jnp.float32),
            pltpu.VMEM((_H,), jnp.float32),
            pltpu.SemaphoreType.DMA,
            pltpu.SemaphoreType.DMA,
            pltpu.SemaphoreType.DMA,
            pltpu.SemaphoreType.DMA,
            pltpu.SemaphoreType.DMA,
        ],
    )
    return f(hl2, hr2, src, dst, ef, w2)


def _scatter_body(msg_hbm, dst_hbm, out_hbm, dst0, rows0, dst1, rows1,
                  zbuf_v, acc_sh, m0, m1, isem):
    cid = lax.axis_index("c")
    sid = lax.axis_index("s")
    row0 = _zero_acc(sid, zbuf_v, acc_sh, 64)
    plsc.subcore_barrier()

    ebase = sid * _EPT

    def load(k, dv, rv, ms):
        c0 = ebase + k * _CS
        ca = pltpu.async_copy(dst_hbm.at[pl.ds(c0, _CS)], dv, isem)
        pltpu.async_copy(msg_hbm.at[pl.ds(cid * _E + c0, _CS)], rv, ms)
        ca.wait()

    def consume(k, dv, rv, ms):
        c0 = ebase + k * _CS
        pltpu.make_async_copy(msg_hbm.at[pl.ds(cid * _E + c0, _CS)],
                              rv, ms).wait()
        pltpu.sync_copy(rv, acc_sh.at[dv], add=True)

    load(0, dst0, rows0, m0)

    def body(g, carry):
        ka = 2 * g
        load(ka + 1, dst1, rows1, m1)
        consume(ka, dst0, rows0, m0)
        load(ka + 2, dst0, rows0, m0)
        consume(ka + 1, dst1, rows1, m1)
        return carry
    lax.fori_loop(0, (_EPT // _CS - 1) // 2, body, 0)
    consume(_EPT // _CS - 1, dst0, rows0, m0)
    plsc.subcore_barrier()
    _readout(cid, sid, row0, acc_sh, out_hbm)


def _sc_scatter(msg2, dst):
    f = pl.kernel(
        _scatter_body,
        out_type=jax.ShapeDtypeStruct((_NC * _N, _H), jnp.float32),
        mesh=_mesh(),
        compiler_params=pltpu.CompilerParams(use_tc_tiling_on_sc=False),
        scratch_types=[
            pltpu.VMEM((_CS,), jnp.int32),
            pltpu.VMEM((_CS, _H), jnp.float32),
            pltpu.VMEM((_CS,), jnp.int32),
            pltpu.VMEM((_CS, _H), jnp.float32),
            pltpu.VMEM((64, _H), jnp.float32),
            pltpu.VMEM_SHARED((_N, _H), jnp.float32),
            pltpu.SemaphoreType.DMA,
            pltpu.SemaphoreType.DMA,
            pltpu.SemaphoreType.DMA,
        ],
    )
    return f(msg2, dst)


def _full(shape):
    return pl.BlockSpec(shape, lambda i: tuple(0 for _ in shape))


def _tc_mlp2(x, W1, b1, W2, b2):
    F = x.shape[1]

    def body(x_ref, w1_ref, b1_ref, w2_ref, b2_ref, o_ref):
        h = jnp.maximum(x_ref[...] @ w1_ref[...] + b1_ref[...], 0.0)
        o_ref[...] = jnp.maximum(h @ w2_ref[...] + b2_ref[...], 0.0)

    return pl.pallas_call(
        body,
        grid=(_NBLK,),
        in_specs=[pl.BlockSpec((_B, F), lambda i: (i, 0)),
                  _full((F, _EMB)), _full((1, _EMB)),
                  _full((_EMB, _EMB)), _full((1, _EMB))],
        out_specs=pl.BlockSpec((_B, _EMB), lambda i: (i, 0)),
        out_shape=jax.ShapeDtypeStruct((_N, _EMB), jnp.float32),
    )(x, W1, b1.reshape(1, -1), W2, b2.reshape(1, -1))


def _tc_mlp2_stacked(x, W1, b1, W2, b2):
    def body(x_ref, w1_ref, b1_ref, w2_ref, b2_ref, o_ref):
        h = jnp.maximum(x_ref[...] @ w1_ref[...] + b1_ref[...], 0.0)
        h2 = jnp.maximum(h @ w2_ref[...] + b2_ref[...], 0.0)
        o_ref[0] = h2[:, :_H]
        o_ref[1] = h2[:, _H:]

    return pl.pallas_call(
        body,
        grid=(_NBLK,),
        in_specs=[pl.BlockSpec((_B, _EMB), lambda i: (i, 0)),
                  _full((_EMB, _EMB)), _full((1, _EMB)),
                  _full((_EMB, _EMB)), _full((1, _EMB))],
        out_specs=pl.BlockSpec((2, _B, _H), lambda i: (0, i, 0)),
        out_shape=jax.ShapeDtypeStruct((2, _N, _H), jnp.float32),
    )(x, W1, b1.reshape(1, -1), W2, b2.reshape(1, -1))


def _tc_bgc_pre(right, left, fml_W, fml_b, fmr_W):
    def body(r_ref, l_ref, wl_ref, bl_ref, wr_ref, hl_ref, hr_ref):
        hl = r_ref[...] @ wl_ref[...] + bl_ref[...]
        hr = l_ref[...] @ wr_ref[...]
        hl_ref[0] = hl[:, :_H]
        hl_ref[1] = hl[:, _H:]
        hr_ref[0] = hr[:, :_H]
        hr_ref[1] = hr[:, _H:]

    return pl.pallas_call(
        body,
        grid=(_NBLK,),
        in_specs=[pl.BlockSpec((_B, _EMB), lambda i: (i, 0)),
                  pl.BlockSpec((_B, _EMB), lambda i: (i, 0)),
                  _full((_EMB, _EMB)), _full((1, _EMB)), _full((_EMB, _EMB))],
        out_specs=[pl.BlockSpec((2, _B, _H), lambda i: (0, i, 0)),
                   pl.BlockSpec((2, _B, _H), lambda i: (0, i, 0))],
        out_shape=[jax.ShapeDtypeStruct((2, _N, _H), jnp.float32),
                   jax.ShapeDtypeStruct((2, _N, _H), jnp.float32)],
    )(right, left, fml_W, fml_b.reshape(1, -1), fmr_W)


def _agg_specs():
    return [pl.BlockSpec((_B, _H), lambda i: (i, 0)),
            pl.BlockSpec((_B, _H), lambda i: (i + _NBLK, 0))]


def _tc_msg(pre2, fmf_W, fmf_b):
    def body(xl_ref, xh_ref, w_ref, b_ref, o_ref):
        x = jnp.concatenate([xl_ref[...], xh_ref[...]], axis=1)
        m = x @ w_ref[...] + b_ref[...]
        o_ref[0] = m[:, :_H]
        o_ref[1] = m[:, _H:]

    return pl.pallas_call(
        body,
        grid=(_EBLK,),
        in_specs=[pl.BlockSpec((_B, _H), lambda i: (i, 0)),
                  pl.BlockSpec((_B, _H), lambda i: (i + _EBLK, 0)),
                  _full((_EMB, _EMB)), _full((1, _EMB))],
        out_specs=pl.BlockSpec((2, _B, _H), lambda i: (0, i, 0)),
        out_shape=jax.ShapeDtypeStruct((2, _E, _H), jnp.float32),
    )(pre2, pre2, fmf_W, fmf_b.reshape(1, -1))


def _tc_bgc_post(agg, right, o1_W, o1_b, o2_W, o2_b):
    def body(al_ref, ah_ref, r_ref, w1_ref, b1_ref, w2_ref, b2_ref, o_ref):
        a = jnp.concatenate([al_ref[...], ah_ref[...]], axis=1)
        h = jnp.concatenate([a, r_ref[...]], axis=1)
        h = jnp.maximum(h @ w1_ref[...] + b1_ref[...], 0.0)
        o_ref[...] = h @ w2_ref[...] + b2_ref[...]

    return pl.pallas_call(
        body,
        grid=(_NBLK,),
        in_specs=_agg_specs() + [
            pl.BlockSpec((_B, _EMB), lambda i: (i, 0)),
            _full((2 * _EMB, _EMB)), _full((1, _EMB)),
            _full((_EMB, _EMB)), _full((1, _EMB))],
        out_specs=pl.BlockSpec((_B, _EMB), lambda i: (i, 0)),
        out_shape=jax.ShapeDtypeStruct((_N, _EMB), jnp.float32),
    )(agg, agg, right, o1_W, o1_b.reshape(1, -1), o2_W, o2_b.reshape(1, -1))


def _tc_layer_post(agg, z, zt, g, beta, jW, jb):
    def body(al_ref, ah_ref, z_ref, zt_ref, g_ref, be_ref, jw_ref,
             bb_ref, nz_ref, zt2_ref):
        gc = jnp.concatenate([al_ref[...], ah_ref[...]], axis=1) + z_ref[...]
        m = jnp.mean(gc, axis=1, keepdims=True)
        d = gc - m
        v = jnp.mean(d * d, axis=1, keepdims=True)
        nz = d / jnp.sqrt(v + 1e-5) * g_ref[...] + be_ref[...]
        nz_ref[...] = nz
        u = jnp.concatenate([nz, zt_ref[...]], axis=1) @ jw_ref[...] + bb_ref[...]
        zt2_ref[...] = 1.0 / (1.0 + jnp.exp(-u))

    return pl.pallas_call(
        body,
        grid=(_NBLK,),
        in_specs=_agg_specs() + [
            pl.BlockSpec((_B, _EMB), lambda i: (i, 0)),
            pl.BlockSpec((_B, _EMB), lambda i: (i, 0)),
            _full((1, _EMB)), _full((1, _EMB)),
            _full((2 * _EMB, _EMB)), _full((1, _EMB))],
        out_specs=[pl.BlockSpec((_B, _EMB), lambda i: (i, 0)),
                   pl.BlockSpec((_B, _EMB), lambda i: (i, 0))],
        out_shape=[jax.ShapeDtypeStruct((_N, _EMB), jnp.float32),
                   jax.ShapeDtypeStruct((_N, _EMB), jnp.float32)],
    )(agg, agg, z, zt, g.reshape(1, -1), beta.reshape(1, -1), jW,
      jb.reshape(1, -1))


def _bgc(left, src, dst, ef, right, p):
    hl2, hr2 = _tc_bgc_pre(right, left, p["fml_W"], p["fml_b"], p["fmr_W"])
    pre2 = _sc_bgc_edge(hl2.reshape(_NC * _N, _H), hr2.reshape(_NC * _N, _H),
                        src, dst, ef, p["fme_W"].reshape(_NC, _H))
    msg2 = _tc_msg(pre2, p["fmf_W"], p["fmf_b"])
    agg = _sc_scatter(msg2.reshape(_NC * _E, _H), dst)
    return _tc_bgc_post(agg, right, p["o1_W"], p["o1_b"], p["o2_W"],
                        p["o2_b"])


def kernel(constraint_features, edge_indices, edge_attrs, variable_features,
           params):
    e0 = edge_indices[0]
    e1 = edge_indices[1]
    ef = edge_attrs[:, 0]
    P = params

    cons = _tc_mlp2(constraint_features, P["ceW1"], P["ceb1"], P["ceW2"],
                    P["ceb2"])
    var = _tc_mlp2(variable_features, P["veW1"], P["veb1"], P["veW2"],
                   P["veb2"])
    cons = _bgc(var, e1, e0, ef, cons, P["v2c"])
    var = _bgc(cons, e0, e1, ef, var, P["c2v"])

    zc, zv, ztc, ztv = cons, var, cons, var
    for p in P["layers"]:
        mc2 = _tc_mlp2_stacked(zc, p["cW1"], p["cb1"], p["cW2"], p["cb2"])
        mv2 = _tc_mlp2_stacked(zv, p["vW1"], p["vb1"], p["vW2"], p["vb2"])
        aggc = _sc_spmm(mv2.reshape(_NC * _N, _H), e1, e0, ef)
        aggv = _sc_spmm(mc2.reshape(_NC * _N, _H), e0, e1, ef)
        zc, ztc = _tc_layer_post(aggc, zc, ztc, p["c_g"], p["c_b"],
                                 p["jcW"], p["jcb"])
        zv, ztv = _tc_layer_post(aggv, zv, ztv, p["v_g"], p["v_b"],
                                 p["jvW"], p["jvb"])
    return zc, zv, ztc, ztv
